# baseline (speedup 1.0000x reference)
import functools

import jax
import jax.numpy as jnp
from jax import lax
from jax.experimental import pallas as pl
from jax.experimental.pallas import tpu as pltpu
from jax.experimental.pallas import tpu_sc as plsc

N = 10000
E = 320000
D = 128
H = 8
C = 16
HC = 128

NC = 2
NS = 16
NW = NC * NS
EB = 40
EPW = E // NW
NBLK = EPW // EB
NP = 10112
RPS = NP // NS

_NBLK_TC = 10
_BN = N // _NBLK_TC


def _proj0_body(x_ref, m_ref, fill_ref, w_ref, wrow_ref, b_ref, sel_ref,
                qkvs_ref, nrm_ref):
    i = pl.program_id(0)
    x = x_ref[...]
    m = m_ref[...]
    x0 = jnp.where(m > 0.5, fill_ref[...], x)
    acc = jnp.dot(x0, w_ref[...], preferred_element_type=jnp.float32)
    acc = acc + m[:, 0:1] * wrow_ref[...]
    acc = acc + b_ref[...]
    qkvs_ref[...] = acc
    qk = acc[:, :256]
    n2 = jnp.dot(qk * qk, sel_ref[...], preferred_element_type=jnp.float32)
    bmax = jnp.max(n2, axis=0, keepdims=True)

    @pl.when(i == 0)
    def _():
        nrm_ref[...] = bmax

    @pl.when(i > 0)
    def _():
        nrm_ref[...] = jnp.maximum(nrm_ref[...], bmax)


def _proj1_body(x_ref, w_ref, b_ref, sel_ref, qkvs_ref, nrm_ref):
    i = pl.program_id(0)
    x = x_ref[...]
    acc = jnp.dot(x, w_ref[...], preferred_element_type=jnp.float32)
    acc = acc + b_ref[...]
    qkvs_ref[...] = acc
    qk = acc[:, :256]
    n2 = jnp.dot(qk * qk, sel_ref[...], preferred_element_type=jnp.float32)
    bmax = jnp.max(n2, axis=0, keepdims=True)

    @pl.when(i == 0)
    def _():
        nrm_ref[...] = bmax

    @pl.when(i > 0)
    def _():
        nrm_ref[...] = jnp.maximum(nrm_ref[...], bmax)


def _proj0_call(x, m128, fill, w, wrow, b, sel):
    return pl.pallas_call(
        _proj0_body,
        grid=(_NBLK_TC,),
        in_specs=[
            pl.BlockSpec((_BN, 128), lambda i: (i, 0)),
            pl.BlockSpec((_BN, 128), lambda i: (i, 0)),
            pl.BlockSpec((1, 128), lambda i: (0, 0)),
            pl.BlockSpec((128, 512), lambda i: (0, 0)),
            pl.BlockSpec((1, 512), lambda i: (0, 0)),
            pl.BlockSpec((1, 512), lambda i: (0, 0)),
            pl.BlockSpec((256, 16), lambda i: (0, 0)),
        ],
        out_specs=[
            pl.BlockSpec((_BN, 512), lambda i: (i, 0)),
            pl.BlockSpec((1, 16), lambda i: (0, 0)),
        ],
        out_shape=[
            jax.ShapeDtypeStruct((N, 512), jnp.float32),
            jax.ShapeDtypeStruct((1, 16), jnp.float32),
        ],
    )(x, m128, fill, w, wrow, b, sel)


def _proj1_call(x, w, b, sel):
    return pl.pallas_call(
        _proj1_body,
        grid=(_NBLK_TC,),
        in_specs=[
            pl.BlockSpec((_BN, 128), lambda i: (i, 0)),
            pl.BlockSpec((128, 512), lambda i: (0, 0)),
            pl.BlockSpec((1, 512), lambda i: (0, 0)),
            pl.BlockSpec((256, 16), lambda i: (0, 0)),
        ],
        out_specs=[
            pl.BlockSpec((_BN, 512), lambda i: (i, 0)),
            pl.BlockSpec((1, 16), lambda i: (0, 0)),
        ],
        out_shape=[
            jax.ShapeDtypeStruct((N, 512), jnp.float32),
            jax.ShapeDtypeStruct((1, 16), jnp.float32),
        ],
    )(x, w, b, sel)


def _edge_body(q_hbm, kv_hbm, dst_hbm, src_hbm, m_hbm, z_hbm,
               acc_out,
               acc_sh, dr0, dr1, dr2, dr3, sr0, sr1, sr2, sr3,
               qrows0, qrows1, kvrows0, kvrows1, wv, exv, mv,
               sem_gq0, sem_gq1, sem_gkv0, sem_gkv1, sem_sc, sem_id, sem_is):
    c = lax.axis_index("c")
    s = lax.axis_index("s")
    wid = c * NS + s

    pltpu.sync_copy(z_hbm.at[pl.ds(s * RPS, RPS)],
                    acc_sh.at[pl.ds(s * RPS, RPS)])
    pltpu.sync_copy(m_hbm, mv)

    plsc.subcore_barrier()

    mvec = mv[...]
    lidx = lax.iota(jnp.int32, 16)
    lane8 = lidx < 8
    ohs = [(lidx == h).astype(jnp.float32) for h in range(H)]
    ebase = wid * EPW
    drs = (dr0, dr1, dr2, dr3)
    srs = (sr0, sr1, sr2, sr3)
    qbufs = (qrows0, qrows1)
    kvbufs = (kvrows0, kvrows1)
    gqsems = (sem_gq0, sem_gq1)
    gkvsems = (sem_gkv0, sem_gkv1)

    pltpu.sync_copy(dst_hbm.at[pl.ds(ebase, EB)], dr0)
    pltpu.sync_copy(src_hbm.at[pl.ds(ebase, EB)], sr0)
    pltpu.sync_copy(dst_hbm.at[pl.ds(ebase + EB, EB)], dr1)
    pltpu.sync_copy(src_hbm.at[pl.ds(ebase + EB, EB)], sr1)
    pltpu.async_copy(q_hbm.at[dr0], qrows0, sem_gq0)
    pltpu.async_copy(kv_hbm.at[sr0], kvrows0, sem_gkv0)
    pltpu.async_copy(q_hbm.at[dr1], qrows1, sem_gq1)
    pltpu.async_copy(kv_hbm.at[sr1], kvrows1, sem_gkv1)

    def do_block(j, quad, prefetch):
        par = quad % 2
        qrows = qbufs[par]
        kvrows = kvbufs[par]
        dcur = drs[quad]

        pltpu.make_async_copy(q_hbm.at[dcur], qrows, gqsems[par]).wait()
        pltpu.make_async_copy(kv_hbm.at[dcur], kvrows, gkvsems[par]).wait()

        if prefetch:
            b2 = ebase + (j + 2) * EB
            pltpu.async_copy(dst_hbm.at[pl.ds(b2, EB)], drs[(quad + 2) % 4],
                             sem_id)
            pltpu.async_copy(src_hbm.at[pl.ds(b2, EB)], srs[(quad + 2) % 4],
                             sem_is)

        @plsc.parallel_loop(0, EB, step=1, unroll=8)
        def _(e):
            svec = jnp.zeros((16,), jnp.float32)
            for h in range(H):
                ph = (qrows[e, pl.ds(h * 16, 16)]
                      * kvrows[e, pl.ds(h * 16, 16)])
                svec = svec + ohs[h] * jnp.sum(ph)
            ex = jnp.exp(svec * 0.25 - mvec)
            exv[e, :] = jnp.where(lane8, ex, 0.0)

        @pl.when(j > 0)
        def _():
            pltpu.make_async_copy(wv, acc_sh.at[dcur], sem_sc).wait()

        @plsc.parallel_loop(0, EB, step=1, unroll=4)
        def _(e):
            ex = exv[e, :]
            wv[e, pl.ds(128, 16)] = ex
            for h in range(H):
                bh = jnp.full((16,), ex[h], jnp.float32)
                wv[e, pl.ds(h * 16, 16)] = (
                    kvrows[e, pl.ds(128 + h * 16, 16)] * bh)

        pltpu.async_copy(wv, acc_sh.at[dcur], sem_sc, add=True)

        if prefetch:
            dnxt = drs[(quad + 2) % 4]
            snxt = srs[(quad + 2) % 4]
            pltpu.make_async_copy(dst_hbm.at[pl.ds(0, EB)], dnxt,
                                  sem_id).wait()
            pltpu.make_async_copy(src_hbm.at[pl.ds(0, EB)], snxt,
                                  sem_is).wait()
            pltpu.async_copy(q_hbm.at[dnxt], qrows, gqsems[par])
            pltpu.async_copy(kv_hbm.at[snxt], kvrows, gkvsems[par])

    def blk4(i4, _):
        for t in range(4):
            do_block(i4 * 4 + t, t, True)
        return 0

    lax.fori_loop(0, (NBLK - 2) // 4, blk4, 0)

    do_block(NBLK - 2, (NBLK - 2) % 4, False)
    do_block(NBLK - 1, (NBLK - 1) % 4, False)

    pltpu.make_async_copy(wv, acc_sh.at[dr0], sem_sc).wait()

    plsc.subcore_barrier()

    pltpu.sync_copy(acc_sh.at[pl.ds(s * RPS, RPS)],
                    acc_out.at[c, pl.ds(s * RPS, RPS)])


def _edge_call(q, kv, dst, src, m16, z144):
    mesh = plsc.VectorSubcoreMesh(core_axis_name="c", subcore_axis_name="s")
    kfn = pl.kernel(
        _edge_body,
        out_type=jax.ShapeDtypeStruct((NC, NP, 144), jnp.float32),
        mesh=mesh,
        compiler_params=pltpu.CompilerParams(needs_layout_passes=False,
                                             use_tc_tiling_on_sc=False),
        scratch_types=[
            pltpu.VMEM_SHARED((NP, 144), jnp.float32),
            pltpu.VMEM((EB,), jnp.int32),
            pltpu.VMEM((EB,), jnp.int32),
            pltpu.VMEM((EB,), jnp.int32),
            pltpu.VMEM((EB,), jnp.int32),
            pltpu.VMEM((EB,), jnp.int32),
            pltpu.VMEM((EB,), jnp.int32),
            pltpu.VMEM((EB,), jnp.int32),
            pltpu.VMEM((EB,), jnp.int32),
            pltpu.VMEM((EB, 128), jnp.float32),
            pltpu.VMEM((EB, 128), jnp.float32),
            pltpu.VMEM((EB, 256), jnp.float32),
            pltpu.VMEM((EB, 256), jnp.float32),
            pltpu.VMEM((EB, 144), jnp.float32),
            pltpu.VMEM((EB, 16), jnp.float32),
            pltpu.VMEM((16,), jnp.float32),
            pltpu.SemaphoreType.DMA,
            pltpu.SemaphoreType.DMA,
            pltpu.SemaphoreType.DMA,
            pltpu.SemaphoreType.DMA,
            pltpu.SemaphoreType.DMA,
            pltpu.SemaphoreType.DMA,
            pltpu.SemaphoreType.DMA,
        ],
    )
    return kfn(q, kv, dst, src, m16, z144)


def _comb_core(acc0, acc1, xr, wba, wbb, ex, g, be, relu):
    num = acc0[:, :128] + acc1[:, :128]
    den = acc0[:, 128:] + acc1[:, 128:]
    den128 = jnp.dot(den, ex, preferred_element_type=jnp.float32)
    safe = jnp.where(den128 > 0.0, den128, 1.0)
    out = jnp.where(den128 > 0.0, num / safe, 0.0)
    bl = jnp.sum(out * wba + xr * wbb, axis=1, keepdims=True)
    beta = 1.0 / (1.0 + jnp.exp(-bl))
    y = beta * xr + (1.0 - beta) * out
    mu = jnp.mean(y, axis=1, keepdims=True)
    var = jnp.mean((y - mu) ** 2, axis=1, keepdims=True)
    yn = (y - mu) / jnp.sqrt(var + 1e-5) * g + be
    if relu:
        yn = jnp.maximum(yn, 0.0)
    return yn


def _combproj_body(acc0_ref, acc1_ref, xr_ref, wba_ref, wbb_ref, ex_ref,
                   g_ref, be_ref, w_ref, b_ref, sel_ref, qkvs_ref, nrm_ref):
    i = pl.program_id(0)
    yn = _comb_core(acc0_ref[...], acc1_ref[...], xr_ref[...], wba_ref[...],
                    wbb_ref[...], ex_ref[...], g_ref[...], be_ref[...], True)
    acc = jnp.dot(yn, w_ref[...], preferred_element_type=jnp.float32)
    acc = acc + b_ref[...]
    qkvs_ref[...] = acc
    qk = acc[:, :256]
    n2 = jnp.dot(qk * qk, sel_ref[...], preferred_element_type=jnp.float32)
    bmax = jnp.max(n2, axis=0, keepdims=True)

    @pl.when(i == 0)
    def _():
        nrm_ref[...] = bmax

    @pl.when(i > 0)
    def _():
        nrm_ref[...] = jnp.maximum(nrm_ref[...], bmax)


def _combproj_call(acc0, acc1, xr, wba, wbb, exmat, g, be, w, b, sel):
    return pl.pallas_call(
        _combproj_body,
        grid=(_NBLK_TC,),
        in_specs=[
            pl.BlockSpec((_BN, 144), lambda i: (i, 0)),
            pl.BlockSpec((_BN, 144), lambda i: (i, 0)),
            pl.BlockSpec((_BN, 128), lambda i: (i, 0)),
            pl.BlockSpec((1, 128), lambda i: (0, 0)),
            pl.BlockSpec((1, 128), lambda i: (0, 0)),
            pl.BlockSpec((16, 128), lambda i: (0, 0)),
            pl.BlockSpec((1, 128), lambda i: (0, 0)),
            pl.BlockSpec((1, 128), lambda i: (0, 0)),
            pl.BlockSpec((128, 512), lambda i: (0, 0)),
            pl.BlockSpec((1, 512), lambda i: (0, 0)),
            pl.BlockSpec((256, 16), lambda i: (0, 0)),
        ],
        out_specs=[
            pl.BlockSpec((_BN, 512), lambda i: (i, 0)),
            pl.BlockSpec((1, 16), lambda i: (0, 0)),
        ],
        out_shape=[
            jax.ShapeDtypeStruct((N, 512), jnp.float32),
            jax.ShapeDtypeStruct((1, 16), jnp.float32),
        ],
    )(acc0, acc1, xr, wba, wbb, exmat, g, be, w, b, sel)


def _comb_body(acc0_ref, acc1_ref, xr_ref, wba_ref, wbb_ref, ex_ref,
               g_ref, be_ref, y_ref):
    y_ref[...] = _comb_core(acc0_ref[...], acc1_ref[...], xr_ref[...],
                            wba_ref[...], wbb_ref[...], ex_ref[...],
                            g_ref[...], be_ref[...], False)


def _comb_call(acc0, acc1, xr, wba, wbb, exmat, g, be):
    return pl.pallas_call(
        _comb_body,
        grid=(_NBLK_TC,),
        in_specs=[
            pl.BlockSpec((_BN, 144), lambda i: (i, 0)),
            pl.BlockSpec((_BN, 144), lambda i: (i, 0)),
            pl.BlockSpec((_BN, 128), lambda i: (i, 0)),
            pl.BlockSpec((1, 128), lambda i: (0, 0)),
            pl.BlockSpec((1, 128), lambda i: (0, 0)),
            pl.BlockSpec((16, 128), lambda i: (0, 0)),
            pl.BlockSpec((1, 128), lambda i: (0, 0)),
            pl.BlockSpec((1, 128), lambda i: (0, 0)),
        ],
        out_specs=pl.BlockSpec((_BN, 128), lambda i: (i, 0)),
        out_shape=jax.ShapeDtypeStruct((N, 128), jnp.float32),
    )(acc0, acc1, xr, wba, wbb, exmat, g, be)


def kernel(x_orig, edge_index, missing_mask_tensor, fill_vec,
           Wq0, bq0, Wk0, bk0, Wv0, bv0, Ws0, bs0, Wb0, g0, be0,
           Wq1, bq1, Wk1, bk1, Wv1, bv1, Ws1, bs1, Wb1, g1, be1):
    src = edge_index[0].astype(jnp.int32)
    dst = edge_index[1].astype(jnp.int32)

    sel = (jnp.arange(256)[:, None] // 16 == jnp.arange(16)[None, :]
           ).astype(jnp.float32)
    exmat = (jnp.arange(16)[:, None] == jnp.arange(128)[None, :] // 16
             ).astype(jnp.float32)
    z144 = jnp.zeros((NP, 144), jnp.float32)
    m128 = jnp.broadcast_to(missing_mask_tensor, (N, 128))

    w0 = jnp.concatenate([Wq0, Wk0, Wv0, Ws0], axis=1)
    b0 = jnp.concatenate([bq0, bk0, bv0, bs0]).reshape(1, 512)
    wba0 = (Wb0[:128, 0] + Wb0[256:, 0]).reshape(1, 128)
    wbb0 = (Wb0[128:256, 0] - Wb0[256:, 0]).reshape(1, 128)

    w1 = jnp.concatenate([Wq1, Wk1, Wv1, Ws1], axis=1)
    b1 = jnp.concatenate([bq1, bk1, bv1, bs1]).reshape(1, 512)
    wba1 = (Wb1[:128, 0] + Wb1[256:, 0]).reshape(1, 128)
    wbb1 = (Wb1[128:256, 0] - Wb1[256:, 0]).reshape(1, 128)

    def emax(nrm2):
        m8 = jnp.sqrt(nrm2[0, :8]) * jnp.sqrt(nrm2[0, 8:]) * 0.25
        return jnp.concatenate([m8, jnp.zeros((8,), jnp.float32)])

    qkvs, nrm2 = _proj0_call(x_orig, m128, fill_vec,
                             w0[:128], w0[128:129].reshape(1, 512), b0, sel)
    acc = _edge_call(qkvs[:, :128], qkvs[:, 128:384], dst, src,
                     emax(nrm2), z144)
    qkvs, nrm2 = _combproj_call(acc[0], acc[1], qkvs[:, 384:],
                                wba0, wbb0, exmat,
                                g0.reshape(1, 128), be0.reshape(1, 128),
                                w1, b1, sel)
    acc = _edge_call(qkvs[:, :128], qkvs[:, 128:384], dst, src,
                     emax(nrm2), z144)
    return _comb_call(acc[0], acc[1], qkvs[:, 384:], wba1, wbb1, exmat,
                      g1.reshape(1, 128), be1.reshape(1, 128))


# speedup vs baseline: 1.0849x; 1.0849x over previous
import functools

import jax
import jax.numpy as jnp
from jax import lax
from jax.experimental import pallas as pl
from jax.experimental.pallas import tpu as pltpu
from jax.experimental.pallas import tpu_sc as plsc

N = 10000
E = 320000
D = 128
H = 8
C = 16
HC = 128

NC = 2
NS = 16
NW = NC * NS
EB = 40
EPW = E // NW
NBLK = EPW // EB
NP = 10112
RPS = NP // NS

_NBLK_TC = 10
_BN = N // _NBLK_TC


def _proj0_body(x_ref, m_ref, fill_ref, w_ref, wrow_ref, b_ref, sel_ref,
                qkvs_ref, nrm_ref):
    i = pl.program_id(0)
    x = x_ref[...]
    m = m_ref[...]
    x0 = jnp.where(m > 0.5, fill_ref[...], x)
    acc = jnp.dot(x0, w_ref[...], preferred_element_type=jnp.float32)
    acc = acc + m[:, 0:1] * wrow_ref[...]
    acc = acc + b_ref[...]
    qkvs_ref[...] = acc
    qk = acc[:, :256]
    n2 = jnp.dot(qk * qk, sel_ref[...], preferred_element_type=jnp.float32)
    bmax = jnp.max(n2, axis=0, keepdims=True)

    @pl.when(i == 0)
    def _():
        nrm_ref[...] = bmax

    @pl.when(i > 0)
    def _():
        nrm_ref[...] = jnp.maximum(nrm_ref[...], bmax)


def _proj1_body(x_ref, w_ref, b_ref, sel_ref, qkvs_ref, nrm_ref):
    i = pl.program_id(0)
    x = x_ref[...]
    acc = jnp.dot(x, w_ref[...], preferred_element_type=jnp.float32)
    acc = acc + b_ref[...]
    qkvs_ref[...] = acc
    qk = acc[:, :256]
    n2 = jnp.dot(qk * qk, sel_ref[...], preferred_element_type=jnp.float32)
    bmax = jnp.max(n2, axis=0, keepdims=True)

    @pl.when(i == 0)
    def _():
        nrm_ref[...] = bmax

    @pl.when(i > 0)
    def _():
        nrm_ref[...] = jnp.maximum(nrm_ref[...], bmax)


def _proj0_call(x, m128, fill, w, wrow, b, sel):
    return pl.pallas_call(
        _proj0_body,
        grid=(_NBLK_TC,),
        in_specs=[
            pl.BlockSpec((_BN, 128), lambda i: (i, 0)),
            pl.BlockSpec((_BN, 128), lambda i: (i, 0)),
            pl.BlockSpec((1, 128), lambda i: (0, 0)),
            pl.BlockSpec((128, 512), lambda i: (0, 0)),
            pl.BlockSpec((1, 512), lambda i: (0, 0)),
            pl.BlockSpec((1, 512), lambda i: (0, 0)),
            pl.BlockSpec((256, 16), lambda i: (0, 0)),
        ],
        out_specs=[
            pl.BlockSpec((_BN, 512), lambda i: (i, 0)),
            pl.BlockSpec((1, 16), lambda i: (0, 0)),
        ],
        out_shape=[
            jax.ShapeDtypeStruct((N, 512), jnp.float32),
            jax.ShapeDtypeStruct((1, 16), jnp.float32),
        ],
    )(x, m128, fill, w, wrow, b, sel)


def _proj1_call(x, w, b, sel):
    return pl.pallas_call(
        _proj1_body,
        grid=(_NBLK_TC,),
        in_specs=[
            pl.BlockSpec((_BN, 128), lambda i: (i, 0)),
            pl.BlockSpec((128, 512), lambda i: (0, 0)),
            pl.BlockSpec((1, 512), lambda i: (0, 0)),
            pl.BlockSpec((256, 16), lambda i: (0, 0)),
        ],
        out_specs=[
            pl.BlockSpec((_BN, 512), lambda i: (i, 0)),
            pl.BlockSpec((1, 16), lambda i: (0, 0)),
        ],
        out_shape=[
            jax.ShapeDtypeStruct((N, 512), jnp.float32),
            jax.ShapeDtypeStruct((1, 16), jnp.float32),
        ],
    )(x, w, b, sel)


def _edge_body(q_hbm, kv_hbm, dst_hbm, src_hbm, m_hbm, z_hbm,
               acc_out,
               acc_sh, dr0, dr1, dr2, dr3, sr0, sr1, sr2, sr3,
               qrows0, qrows1, kvrows0, kvrows1, wv, exv, mv,
               sem_gq0, sem_gq1, sem_gkv0, sem_gkv1, sem_sc, sem_id, sem_is):
    c = lax.axis_index("c")
    s = lax.axis_index("s")
    wid = c * NS + s

    pltpu.sync_copy(z_hbm.at[pl.ds(s * RPS, RPS)],
                    acc_sh.at[pl.ds(s * RPS, RPS)])
    pltpu.sync_copy(m_hbm, mv)

    plsc.subcore_barrier()

    mvec = mv[...]
    lidx = lax.iota(jnp.int32, 16)
    lane8 = lidx < 8
    ohs = [(lidx == h).astype(jnp.float32) for h in range(H)]
    ebase = wid * EPW
    drs = (dr0, dr1, dr2, dr3)
    srs = (sr0, sr1, sr2, sr3)
    qbufs = (qrows0, qrows1)
    kvbufs = (kvrows0, kvrows1)
    gqsems = (sem_gq0, sem_gq1)
    gkvsems = (sem_gkv0, sem_gkv1)

    pltpu.sync_copy(dst_hbm.at[pl.ds(ebase, EB)], dr0)
    pltpu.sync_copy(src_hbm.at[pl.ds(ebase, EB)], sr0)
    pltpu.sync_copy(dst_hbm.at[pl.ds(ebase + EB, EB)], dr1)
    pltpu.sync_copy(src_hbm.at[pl.ds(ebase + EB, EB)], sr1)
    pltpu.async_copy(q_hbm.at[dr0], qrows0, sem_gq0)
    pltpu.async_copy(kv_hbm.at[sr0], kvrows0, sem_gkv0)
    pltpu.async_copy(q_hbm.at[dr1], qrows1, sem_gq1)
    pltpu.async_copy(kv_hbm.at[sr1], kvrows1, sem_gkv1)

    def do_block(j, quad, prefetch):
        par = quad % 2
        qrows = qbufs[par]
        kvrows = kvbufs[par]
        dcur = drs[quad]

        pltpu.make_async_copy(q_hbm.at[dcur], qrows, gqsems[par]).wait()
        pltpu.make_async_copy(kv_hbm.at[dcur], kvrows, gkvsems[par]).wait()

        if prefetch:
            b2 = ebase + (j + 2) * EB
            pltpu.async_copy(dst_hbm.at[pl.ds(b2, EB)], drs[(quad + 2) % 4],
                             sem_id)
            pltpu.async_copy(src_hbm.at[pl.ds(b2, EB)], srs[(quad + 2) % 4],
                             sem_is)

        @plsc.parallel_loop(0, EB, step=1, unroll=5)
        def _(e):
            svec = jnp.zeros((16,), jnp.float32)
            for h in range(H):
                ph = (qrows[e, pl.ds(h * 16, 16)]
                      * kvrows[e, pl.ds(h * 16, 16)])
                svec = svec + ohs[h] * jnp.sum(ph)
            ex = jnp.exp(svec * 0.25 - mvec)
            exv[e, :] = jnp.where(lane8, ex, 0.0)

        @pl.when(j > 0)
        def _():
            pltpu.make_async_copy(wv, acc_sh.at[dcur], sem_sc).wait()

        @plsc.parallel_loop(0, EB, step=1, unroll=4)
        def _(e):
            ex = exv[e, :]
            wv[e, pl.ds(128, 16)] = ex
            for h in range(H):
                bh = jnp.full((16,), ex[h], jnp.float32)
                wv[e, pl.ds(h * 16, 16)] = (
                    kvrows[e, pl.ds(128 + h * 16, 16)] * bh)

        pltpu.async_copy(wv, acc_sh.at[dcur], sem_sc, add=True)

        if prefetch:
            dnxt = drs[(quad + 2) % 4]
            snxt = srs[(quad + 2) % 4]
            pltpu.make_async_copy(dst_hbm.at[pl.ds(0, EB)], dnxt,
                                  sem_id).wait()
            pltpu.make_async_copy(src_hbm.at[pl.ds(0, EB)], snxt,
                                  sem_is).wait()
            pltpu.async_copy(q_hbm.at[dnxt], qrows, gqsems[par])
            pltpu.async_copy(kv_hbm.at[snxt], kvrows, gkvsems[par])

    def blk4(i4, _):
        for t in range(4):
            do_block(i4 * 4 + t, t, True)
        return 0

    lax.fori_loop(0, (NBLK - 2) // 4, blk4, 0)

    do_block(NBLK - 2, (NBLK - 2) % 4, False)
    do_block(NBLK - 1, (NBLK - 1) % 4, False)

    pltpu.make_async_copy(wv, acc_sh.at[dr0], sem_sc).wait()

    plsc.subcore_barrier()

    pltpu.sync_copy(acc_sh.at[pl.ds(s * RPS, RPS)],
                    acc_out.at[c, pl.ds(s * RPS, RPS)])


def _edge_call(q, kv, dst, src, m16, z144):
    mesh = plsc.VectorSubcoreMesh(core_axis_name="c", subcore_axis_name="s")
    kfn = pl.kernel(
        _edge_body,
        out_type=jax.ShapeDtypeStruct((NC, NP, 144), jnp.float32),
        mesh=mesh,
        compiler_params=pltpu.CompilerParams(needs_layout_passes=False,
                                             use_tc_tiling_on_sc=False),
        scratch_types=[
            pltpu.VMEM_SHARED((NP, 144), jnp.float32),
            pltpu.VMEM((EB,), jnp.int32),
            pltpu.VMEM((EB,), jnp.int32),
            pltpu.VMEM((EB,), jnp.int32),
            pltpu.VMEM((EB,), jnp.int32),
            pltpu.VMEM((EB,), jnp.int32),
            pltpu.VMEM((EB,), jnp.int32),
            pltpu.VMEM((EB,), jnp.int32),
            pltpu.VMEM((EB,), jnp.int32),
            pltpu.VMEM((EB, 128), jnp.float32),
            pltpu.VMEM((EB, 128), jnp.float32),
            pltpu.VMEM((EB, 256), jnp.float32),
            pltpu.VMEM((EB, 256), jnp.float32),
            pltpu.VMEM((EB, 144), jnp.float32),
            pltpu.VMEM((EB, 16), jnp.float32),
            pltpu.VMEM((16,), jnp.float32),
            pltpu.SemaphoreType.DMA,
            pltpu.SemaphoreType.DMA,
            pltpu.SemaphoreType.DMA,
            pltpu.SemaphoreType.DMA,
            pltpu.SemaphoreType.DMA,
            pltpu.SemaphoreType.DMA,
            pltpu.SemaphoreType.DMA,
        ],
    )
    return kfn(q, kv, dst, src, m16, z144)


def _comb_core(acc0, acc1, xr, wba, wbb, ex, g, be, relu):
    num = acc0[:, :128] + acc1[:, :128]
    den = acc0[:, 128:] + acc1[:, 128:]
    den128 = jnp.dot(den, ex, preferred_element_type=jnp.float32)
    safe = jnp.where(den128 > 0.0, den128, 1.0)
    out = jnp.where(den128 > 0.0, num / safe, 0.0)
    bl = jnp.sum(out * wba + xr * wbb, axis=1, keepdims=True)
    beta = 1.0 / (1.0 + jnp.exp(-bl))
    y = beta * xr + (1.0 - beta) * out
    mu = jnp.mean(y, axis=1, keepdims=True)
    var = jnp.mean((y - mu) ** 2, axis=1, keepdims=True)
    yn = (y - mu) / jnp.sqrt(var + 1e-5) * g + be
    if relu:
        yn = jnp.maximum(yn, 0.0)
    return yn


def _combproj_body(acc0_ref, acc1_ref, xr_ref, wba_ref, wbb_ref, ex_ref,
                   g_ref, be_ref, w_ref, b_ref, sel_ref, qkvs_ref, nrm_ref):
    i = pl.program_id(0)
    yn = _comb_core(acc0_ref[...], acc1_ref[...], xr_ref[...], wba_ref[...],
                    wbb_ref[...], ex_ref[...], g_ref[...], be_ref[...], True)
    acc = jnp.dot(yn, w_ref[...], preferred_element_type=jnp.float32)
    acc = acc + b_ref[...]
    qkvs_ref[...] = acc
    qk = acc[:, :256]
    n2 = jnp.dot(qk * qk, sel_ref[...], preferred_element_type=jnp.float32)
    bmax = jnp.max(n2, axis=0, keepdims=True)

    @pl.when(i == 0)
    def _():
        nrm_ref[...] = bmax

    @pl.when(i > 0)
    def _():
        nrm_ref[...] = jnp.maximum(nrm_ref[...], bmax)


def _combproj_call(acc0, acc1, xr, wba, wbb, exmat, g, be, w, b, sel):
    return pl.pallas_call(
        _combproj_body,
        grid=(_NBLK_TC,),
        in_specs=[
            pl.BlockSpec((_BN, 144), lambda i: (i, 0)),
            pl.BlockSpec((_BN, 144), lambda i: (i, 0)),
            pl.BlockSpec((_BN, 128), lambda i: (i, 0)),
            pl.BlockSpec((1, 128), lambda i: (0, 0)),
            pl.BlockSpec((1, 128), lambda i: (0, 0)),
            pl.BlockSpec((16, 128), lambda i: (0, 0)),
            pl.BlockSpec((1, 128), lambda i: (0, 0)),
            pl.BlockSpec((1, 128), lambda i: (0, 0)),
            pl.BlockSpec((128, 512), lambda i: (0, 0)),
            pl.BlockSpec((1, 512), lambda i: (0, 0)),
            pl.BlockSpec((256, 16), lambda i: (0, 0)),
        ],
        out_specs=[
            pl.BlockSpec((_BN, 512), lambda i: (i, 0)),
            pl.BlockSpec((1, 16), lambda i: (0, 0)),
        ],
        out_shape=[
            jax.ShapeDtypeStruct((N, 512), jnp.float32),
            jax.ShapeDtypeStruct((1, 16), jnp.float32),
        ],
    )(acc0, acc1, xr, wba, wbb, exmat, g, be, w, b, sel)


def _comb_body(acc0_ref, acc1_ref, xr_ref, wba_ref, wbb_ref, ex_ref,
               g_ref, be_ref, y_ref):
    y_ref[...] = _comb_core(acc0_ref[...], acc1_ref[...], xr_ref[...],
                            wba_ref[...], wbb_ref[...], ex_ref[...],
                            g_ref[...], be_ref[...], False)


def _comb_call(acc0, acc1, xr, wba, wbb, exmat, g, be):
    return pl.pallas_call(
        _comb_body,
        grid=(_NBLK_TC,),
        in_specs=[
            pl.BlockSpec((_BN, 144), lambda i: (i, 0)),
            pl.BlockSpec((_BN, 144), lambda i: (i, 0)),
            pl.BlockSpec((_BN, 128), lambda i: (i, 0)),
            pl.BlockSpec((1, 128), lambda i: (0, 0)),
            pl.BlockSpec((1, 128), lambda i: (0, 0)),
            pl.BlockSpec((16, 128), lambda i: (0, 0)),
            pl.BlockSpec((1, 128), lambda i: (0, 0)),
            pl.BlockSpec((1, 128), lambda i: (0, 0)),
        ],
        out_specs=pl.BlockSpec((_BN, 128), lambda i: (i, 0)),
        out_shape=jax.ShapeDtypeStruct((N, 128), jnp.float32),
    )(acc0, acc1, xr, wba, wbb, exmat, g, be)


def kernel(x_orig, edge_index, missing_mask_tensor, fill_vec,
           Wq0, bq0, Wk0, bk0, Wv0, bv0, Ws0, bs0, Wb0, g0, be0,
           Wq1, bq1, Wk1, bk1, Wv1, bv1, Ws1, bs1, Wb1, g1, be1):
    src = edge_index[0].astype(jnp.int32)
    dst = edge_index[1].astype(jnp.int32)

    sel = (jnp.arange(256)[:, None] // 16 == jnp.arange(16)[None, :]
           ).astype(jnp.float32)
    exmat = (jnp.arange(16)[:, None] == jnp.arange(128)[None, :] // 16
             ).astype(jnp.float32)
    z144 = jnp.zeros((NP, 144), jnp.float32)
    m128 = jnp.broadcast_to(missing_mask_tensor, (N, 128))

    w0 = jnp.concatenate([Wq0, Wk0, Wv0, Ws0], axis=1)
    b0 = jnp.concatenate([bq0, bk0, bv0, bs0]).reshape(1, 512)
    wba0 = (Wb0[:128, 0] + Wb0[256:, 0]).reshape(1, 128)
    wbb0 = (Wb0[128:256, 0] - Wb0[256:, 0]).reshape(1, 128)

    w1 = jnp.concatenate([Wq1, Wk1, Wv1, Ws1], axis=1)
    b1 = jnp.concatenate([bq1, bk1, bv1, bs1]).reshape(1, 512)
    wba1 = (Wb1[:128, 0] + Wb1[256:, 0]).reshape(1, 128)
    wbb1 = (Wb1[128:256, 0] - Wb1[256:, 0]).reshape(1, 128)

    def emax(nrm2):
        m8 = jnp.sqrt(nrm2[0, :8]) * jnp.sqrt(nrm2[0, 8:]) * 0.25
        return jnp.concatenate([m8, jnp.zeros((8,), jnp.float32)])

    qkvs, nrm2 = _proj0_call(x_orig, m128, fill_vec,
                             w0[:128], w0[128:129].reshape(1, 512), b0, sel)
    acc = _edge_call(qkvs[:, :128], qkvs[:, 128:384], dst, src,
                     emax(nrm2), z144)
    qkvs, nrm2 = _combproj_call(acc[0], acc[1], qkvs[:, 384:],
                                wba0, wbb0, exmat,
                                g0.reshape(1, 128), be0.reshape(1, 128),
                                w1, b1, sel)
    acc = _edge_call(qkvs[:, :128], qkvs[:, 128:384], dst, src,
                     emax(nrm2), z144)
    return _comb_call(acc[0], acc[1], qkvs[:, 384:], wba1, wbb1, exmat,
                      g1.reshape(1, 128), be1.reshape(1, 128))


# speedup vs baseline: 1.1230x; 1.0351x over previous
import functools

import jax
import jax.numpy as jnp
from jax import lax
from jax.experimental import pallas as pl
from jax.experimental.pallas import tpu as pltpu
from jax.experimental.pallas import tpu_sc as plsc

N = 10000
E = 320000
D = 128
H = 8
C = 16
HC = 128

NC = 2
NS = 16
NW = NC * NS
EB = 40
EPW = E // NW
NBLK = EPW // EB
NP = 10112
RPS = NP // NS

_NBLK_TC = 10
_BN = N // _NBLK_TC


def _proj0_body(x_ref, m_ref, fill_ref, w_ref, wrow_ref, b_ref, sel_ref,
                qkvs_ref, nrm_ref):
    i = pl.program_id(0)
    x = x_ref[...]
    m = m_ref[...]
    x0 = jnp.where(m > 0.5, fill_ref[...], x)
    acc = jnp.dot(x0, w_ref[...], preferred_element_type=jnp.float32)
    acc = acc + m[:, 0:1] * wrow_ref[...]
    acc = acc + b_ref[...]
    qkvs_ref[...] = acc
    qk = acc[:, :256]
    n2 = jnp.dot(qk * qk, sel_ref[...], preferred_element_type=jnp.float32)
    bmax = jnp.max(n2, axis=0, keepdims=True)

    @pl.when(i == 0)
    def _():
        nrm_ref[...] = bmax

    @pl.when(i > 0)
    def _():
        nrm_ref[...] = jnp.maximum(nrm_ref[...], bmax)


def _proj1_body(x_ref, w_ref, b_ref, sel_ref, qkvs_ref, nrm_ref):
    i = pl.program_id(0)
    x = x_ref[...]
    acc = jnp.dot(x, w_ref[...], preferred_element_type=jnp.float32)
    acc = acc + b_ref[...]
    qkvs_ref[...] = acc
    qk = acc[:, :256]
    n2 = jnp.dot(qk * qk, sel_ref[...], preferred_element_type=jnp.float32)
    bmax = jnp.max(n2, axis=0, keepdims=True)

    @pl.when(i == 0)
    def _():
        nrm_ref[...] = bmax

    @pl.when(i > 0)
    def _():
        nrm_ref[...] = jnp.maximum(nrm_ref[...], bmax)


def _proj0_call(x, m128, fill, w, wrow, b, sel):
    return pl.pallas_call(
        _proj0_body,
        grid=(_NBLK_TC,),
        in_specs=[
            pl.BlockSpec((_BN, 128), lambda i: (i, 0)),
            pl.BlockSpec((_BN, 128), lambda i: (i, 0)),
            pl.BlockSpec((1, 128), lambda i: (0, 0)),
            pl.BlockSpec((128, 512), lambda i: (0, 0)),
            pl.BlockSpec((1, 512), lambda i: (0, 0)),
            pl.BlockSpec((1, 512), lambda i: (0, 0)),
            pl.BlockSpec((256, 16), lambda i: (0, 0)),
        ],
        out_specs=[
            pl.BlockSpec((_BN, 512), lambda i: (i, 0)),
            pl.BlockSpec((1, 16), lambda i: (0, 0)),
        ],
        out_shape=[
            jax.ShapeDtypeStruct((N, 512), jnp.float32),
            jax.ShapeDtypeStruct((1, 16), jnp.float32),
        ],
    )(x, m128, fill, w, wrow, b, sel)


def _proj1_call(x, w, b, sel):
    return pl.pallas_call(
        _proj1_body,
        grid=(_NBLK_TC,),
        in_specs=[
            pl.BlockSpec((_BN, 128), lambda i: (i, 0)),
            pl.BlockSpec((128, 512), lambda i: (0, 0)),
            pl.BlockSpec((1, 512), lambda i: (0, 0)),
            pl.BlockSpec((256, 16), lambda i: (0, 0)),
        ],
        out_specs=[
            pl.BlockSpec((_BN, 512), lambda i: (i, 0)),
            pl.BlockSpec((1, 16), lambda i: (0, 0)),
        ],
        out_shape=[
            jax.ShapeDtypeStruct((N, 512), jnp.float32),
            jax.ShapeDtypeStruct((1, 16), jnp.float32),
        ],
    )(x, w, b, sel)


def _edge_body(q_hbm, kv_hbm, dst_hbm, src_hbm, m_hbm, z_hbm,
               acc_out,
               acc_sh, dr0, dr1, dr2, dr3, sr0, sr1, sr2, sr3,
               qrows0, qrows1, kvrows0, kvrows1, wv, exv, mv,
               sem_gq0, sem_gq1, sem_gkv0, sem_gkv1, sem_sc, sem_id, sem_is):
    c = lax.axis_index("c")
    s = lax.axis_index("s")
    wid = c * NS + s

    pltpu.sync_copy(z_hbm.at[pl.ds(s * RPS, RPS)],
                    acc_sh.at[pl.ds(s * RPS, RPS)])
    pltpu.sync_copy(m_hbm, mv)

    plsc.subcore_barrier()

    mvec = mv[...]
    lidx = lax.iota(jnp.int32, 16)
    lane8 = lidx < 8
    ohs = [(lidx == h).astype(jnp.float32) for h in range(H)]
    ebase = wid * EPW
    drs = (dr0, dr1, dr2, dr3)
    srs = (sr0, sr1, sr2, sr3)
    qbufs = (qrows0, qrows1)
    kvbufs = (kvrows0, kvrows1)
    gqsems = (sem_gq0, sem_gq1)
    gkvsems = (sem_gkv0, sem_gkv1)

    pltpu.sync_copy(dst_hbm.at[pl.ds(ebase, EB)], dr0)
    pltpu.sync_copy(src_hbm.at[pl.ds(ebase, EB)], sr0)
    pltpu.sync_copy(dst_hbm.at[pl.ds(ebase + EB, EB)], dr1)
    pltpu.sync_copy(src_hbm.at[pl.ds(ebase + EB, EB)], sr1)
    pltpu.async_copy(q_hbm.at[dr0], qrows0, sem_gq0)
    pltpu.async_copy(kv_hbm.at[sr0], kvrows0, sem_gkv0)
    pltpu.async_copy(q_hbm.at[dr1], qrows1, sem_gq1)
    pltpu.async_copy(kv_hbm.at[sr1], kvrows1, sem_gkv1)

    def do_block(j, quad, prefetch):
        par = quad % 2
        qrows = qbufs[par]
        kvrows = kvbufs[par]
        dcur = drs[quad]

        pltpu.make_async_copy(q_hbm.at[dcur], qrows, gqsems[par]).wait()
        pltpu.make_async_copy(kv_hbm.at[dcur], kvrows, gkvsems[par]).wait()

        if prefetch:
            b2 = ebase + (j + 2) * EB
            pltpu.async_copy(dst_hbm.at[pl.ds(b2, EB)], drs[(quad + 2) % 4],
                             sem_id)
            pltpu.async_copy(src_hbm.at[pl.ds(b2, EB)], srs[(quad + 2) % 4],
                             sem_is)

        @plsc.parallel_loop(0, EB, step=1, unroll=4)
        def _(e):
            svec = jnp.zeros((16,), jnp.float32)
            for h in range(H):
                ph = (qrows[e, pl.ds(h * 16, 16)]
                      * kvrows[e, pl.ds(h * 16, 16)])
                svec = svec + ohs[h] * jnp.sum(ph)
            ex = jnp.exp(svec * 0.25 - mvec)
            exv[e, :] = jnp.where(lane8, ex, 0.0)

        @pl.when(j > 0)
        def _():
            pltpu.make_async_copy(wv, acc_sh.at[dcur], sem_sc).wait()

        @plsc.parallel_loop(0, EB, step=1, unroll=4)
        def _(e):
            ex = exv[e, :]
            wv[e, pl.ds(128, 16)] = ex
            for h in range(H):
                bh = jnp.full((16,), ex[h], jnp.float32)
                wv[e, pl.ds(h * 16, 16)] = (
                    kvrows[e, pl.ds(128 + h * 16, 16)] * bh)

        pltpu.async_copy(wv, acc_sh.at[dcur], sem_sc, add=True)

        if prefetch:
            dnxt = drs[(quad + 2) % 4]
            snxt = srs[(quad + 2) % 4]
            pltpu.make_async_copy(dst_hbm.at[pl.ds(0, EB)], dnxt,
                                  sem_id).wait()
            pltpu.make_async_copy(src_hbm.at[pl.ds(0, EB)], snxt,
                                  sem_is).wait()
            pltpu.async_copy(q_hbm.at[dnxt], qrows, gqsems[par])
            pltpu.async_copy(kv_hbm.at[snxt], kvrows, gkvsems[par])

    def blk4(i4, _):
        for t in range(4):
            do_block(i4 * 4 + t, t, True)
        return 0

    lax.fori_loop(0, (NBLK - 2) // 4, blk4, 0)

    do_block(NBLK - 2, (NBLK - 2) % 4, False)
    do_block(NBLK - 1, (NBLK - 1) % 4, False)

    pltpu.make_async_copy(wv, acc_sh.at[dr0], sem_sc).wait()

    plsc.subcore_barrier()

    pltpu.sync_copy(acc_sh.at[pl.ds(s * RPS, RPS)],
                    acc_out.at[c, pl.ds(s * RPS, RPS)])


def _edge_call(q, kv, dst, src, m16, z144):
    mesh = plsc.VectorSubcoreMesh(core_axis_name="c", subcore_axis_name="s")
    kfn = pl.kernel(
        _edge_body,
        out_type=jax.ShapeDtypeStruct((NC, NP, 144), jnp.float32),
        mesh=mesh,
        compiler_params=pltpu.CompilerParams(needs_layout_passes=False,
                                             use_tc_tiling_on_sc=False),
        scratch_types=[
            pltpu.VMEM_SHARED((NP, 144), jnp.float32),
            pltpu.VMEM((EB,), jnp.int32),
            pltpu.VMEM((EB,), jnp.int32),
            pltpu.VMEM((EB,), jnp.int32),
            pltpu.VMEM((EB,), jnp.int32),
            pltpu.VMEM((EB,), jnp.int32),
            pltpu.VMEM((EB,), jnp.int32),
            pltpu.VMEM((EB,), jnp.int32),
            pltpu.VMEM((EB,), jnp.int32),
            pltpu.VMEM((EB, 128), jnp.float32),
            pltpu.VMEM((EB, 128), jnp.float32),
            pltpu.VMEM((EB, 256), jnp.float32),
            pltpu.VMEM((EB, 256), jnp.float32),
            pltpu.VMEM((EB, 144), jnp.float32),
            pltpu.VMEM((EB, 16), jnp.float32),
            pltpu.VMEM((16,), jnp.float32),
            pltpu.SemaphoreType.DMA,
            pltpu.SemaphoreType.DMA,
            pltpu.SemaphoreType.DMA,
            pltpu.SemaphoreType.DMA,
            pltpu.SemaphoreType.DMA,
            pltpu.SemaphoreType.DMA,
            pltpu.SemaphoreType.DMA,
        ],
    )
    return kfn(q, kv, dst, src, m16, z144)


def _comb_core(acc0, acc1, xr, wba, wbb, ex, g, be, relu):
    num = acc0[:, :128] + acc1[:, :128]
    den = acc0[:, 128:] + acc1[:, 128:]
    den128 = jnp.dot(den, ex, preferred_element_type=jnp.float32)
    safe = jnp.where(den128 > 0.0, den128, 1.0)
    out = jnp.where(den128 > 0.0, num / safe, 0.0)
    bl = jnp.sum(out * wba + xr * wbb, axis=1, keepdims=True)
    beta = 1.0 / (1.0 + jnp.exp(-bl))
    y = beta * xr + (1.0 - beta) * out
    mu = jnp.mean(y, axis=1, keepdims=True)
    var = jnp.mean((y - mu) ** 2, axis=1, keepdims=True)
    yn = (y - mu) / jnp.sqrt(var + 1e-5) * g + be
    if relu:
        yn = jnp.maximum(yn, 0.0)
    return yn


def _combproj_body(acc0_ref, acc1_ref, xr_ref, wba_ref, wbb_ref, ex_ref,
                   g_ref, be_ref, w_ref, b_ref, sel_ref, qkvs_ref, nrm_ref):
    i = pl.program_id(0)
    yn = _comb_core(acc0_ref[...], acc1_ref[...], xr_ref[...], wba_ref[...],
                    wbb_ref[...], ex_ref[...], g_ref[...], be_ref[...], True)
    acc = jnp.dot(yn, w_ref[...], preferred_element_type=jnp.float32)
    acc = acc + b_ref[...]
    qkvs_ref[...] = acc
    qk = acc[:, :256]
    n2 = jnp.dot(qk * qk, sel_ref[...], preferred_element_type=jnp.float32)
    bmax = jnp.max(n2, axis=0, keepdims=True)

    @pl.when(i == 0)
    def _():
        nrm_ref[...] = bmax

    @pl.when(i > 0)
    def _():
        nrm_ref[...] = jnp.maximum(nrm_ref[...], bmax)


def _combproj_call(acc0, acc1, xr, wba, wbb, exmat, g, be, w, b, sel):
    return pl.pallas_call(
        _combproj_body,
        grid=(_NBLK_TC,),
        in_specs=[
            pl.BlockSpec((_BN, 144), lambda i: (i, 0)),
            pl.BlockSpec((_BN, 144), lambda i: (i, 0)),
            pl.BlockSpec((_BN, 128), lambda i: (i, 0)),
            pl.BlockSpec((1, 128), lambda i: (0, 0)),
            pl.BlockSpec((1, 128), lambda i: (0, 0)),
            pl.BlockSpec((16, 128), lambda i: (0, 0)),
            pl.BlockSpec((1, 128), lambda i: (0, 0)),
            pl.BlockSpec((1, 128), lambda i: (0, 0)),
            pl.BlockSpec((128, 512), lambda i: (0, 0)),
            pl.BlockSpec((1, 512), lambda i: (0, 0)),
            pl.BlockSpec((256, 16), lambda i: (0, 0)),
        ],
        out_specs=[
            pl.BlockSpec((_BN, 512), lambda i: (i, 0)),
            pl.BlockSpec((1, 16), lambda i: (0, 0)),
        ],
        out_shape=[
            jax.ShapeDtypeStruct((N, 512), jnp.float32),
            jax.ShapeDtypeStruct((1, 16), jnp.float32),
        ],
    )(acc0, acc1, xr, wba, wbb, exmat, g, be, w, b, sel)


def _comb_body(acc0_ref, acc1_ref, xr_ref, wba_ref, wbb_ref, ex_ref,
               g_ref, be_ref, y_ref):
    y_ref[...] = _comb_core(acc0_ref[...], acc1_ref[...], xr_ref[...],
                            wba_ref[...], wbb_ref[...], ex_ref[...],
                            g_ref[...], be_ref[...], False)


def _comb_call(acc0, acc1, xr, wba, wbb, exmat, g, be):
    return pl.pallas_call(
        _comb_body,
        grid=(_NBLK_TC,),
        in_specs=[
            pl.BlockSpec((_BN, 144), lambda i: (i, 0)),
            pl.BlockSpec((_BN, 144), lambda i: (i, 0)),
            pl.BlockSpec((_BN, 128), lambda i: (i, 0)),
            pl.BlockSpec((1, 128), lambda i: (0, 0)),
            pl.BlockSpec((1, 128), lambda i: (0, 0)),
            pl.BlockSpec((16, 128), lambda i: (0, 0)),
            pl.BlockSpec((1, 128), lambda i: (0, 0)),
            pl.BlockSpec((1, 128), lambda i: (0, 0)),
        ],
        out_specs=pl.BlockSpec((_BN, 128), lambda i: (i, 0)),
        out_shape=jax.ShapeDtypeStruct((N, 128), jnp.float32),
    )(acc0, acc1, xr, wba, wbb, exmat, g, be)


def kernel(x_orig, edge_index, missing_mask_tensor, fill_vec,
           Wq0, bq0, Wk0, bk0, Wv0, bv0, Ws0, bs0, Wb0, g0, be0,
           Wq1, bq1, Wk1, bk1, Wv1, bv1, Ws1, bs1, Wb1, g1, be1):
    src = edge_index[0].astype(jnp.int32)
    dst = edge_index[1].astype(jnp.int32)

    sel = (jnp.arange(256)[:, None] // 16 == jnp.arange(16)[None, :]
           ).astype(jnp.float32)
    exmat = (jnp.arange(16)[:, None] == jnp.arange(128)[None, :] // 16
             ).astype(jnp.float32)
    z144 = jnp.zeros((NP, 144), jnp.float32)
    m128 = jnp.broadcast_to(missing_mask_tensor, (N, 128))

    w0 = jnp.concatenate([Wq0, Wk0, Wv0, Ws0], axis=1)
    b0 = jnp.concatenate([bq0, bk0, bv0, bs0]).reshape(1, 512)
    wba0 = (Wb0[:128, 0] + Wb0[256:, 0]).reshape(1, 128)
    wbb0 = (Wb0[128:256, 0] - Wb0[256:, 0]).reshape(1, 128)

    w1 = jnp.concatenate([Wq1, Wk1, Wv1, Ws1], axis=1)
    b1 = jnp.concatenate([bq1, bk1, bv1, bs1]).reshape(1, 512)
    wba1 = (Wb1[:128, 0] + Wb1[256:, 0]).reshape(1, 128)
    wbb1 = (Wb1[128:256, 0] - Wb1[256:, 0]).reshape(1, 128)

    def emax(nrm2):
        m8 = jnp.sqrt(nrm2[0, :8]) * jnp.sqrt(nrm2[0, 8:]) * 0.25
        return jnp.concatenate([m8, jnp.zeros((8,), jnp.float32)])

    qkvs, nrm2 = _proj0_call(x_orig, m128, fill_vec,
                             w0[:128], w0[128:129].reshape(1, 512), b0, sel)
    acc = _edge_call(qkvs[:, :128], qkvs[:, 128:384], dst, src,
                     emax(nrm2), z144)
    qkvs, nrm2 = _combproj_call(acc[0], acc[1], qkvs[:, 384:],
                                wba0, wbb0, exmat,
                                g0.reshape(1, 128), be0.reshape(1, 128),
                                w1, b1, sel)
    acc = _edge_call(qkvs[:, :128], qkvs[:, 128:384], dst, src,
                     emax(nrm2), z144)
    return _comb_call(acc[0], acc[1], qkvs[:, 384:], wba1, wbb1, exmat,
                      g1.reshape(1, 128), be1.reshape(1, 128))
